# baseline (speedup 1.0000x reference)
import functools

import jax
import jax.numpy as jnp
from jax import lax
from jax.experimental import pallas as pl
from jax.experimental.pallas import tpu as pltpu
from jax.experimental.pallas import tpu_sc as plsc

_KEEP_PERCENT = 0.1
_SCALE = 1.0 / _KEEP_PERCENT
_DROP_PERCENT = 1.0 - _KEEP_PERCENT
_MIN32 = -(2 ** 31)
_LOW31 = 0x7FFFFFFF


def _to_i32(v):
    v &= 0xFFFFFFFF
    return v - 2 ** 32 if v >= 2 ** 31 else v


def _u_of_bits(b):
    m = b ^ (lax.shift_right_arithmetic(b, 31) & _LOW31)
    return m ^ jnp.int32(_MIN32)


def _sc_select(num_samples, n_per_batch, bsz, smap_flat):
    info = plsc.get_sparse_core_info()
    nc, ns = info.num_cores, info.num_subcores
    bpc = bsz // nc
    npt = n_per_batch // ns
    nv = npt // 16
    hb = bpc * 256
    mesh = plsc.VectorSubcoreMesh(core_axis_name="c", subcore_axis_name="s")

    @functools.partial(
        pl.kernel,
        mesh=mesh,
        compiler_params=pltpu.CompilerParams(needs_layout_passes=False),
        out_type=(
            jax.ShapeDtypeStruct((bsz * n_per_batch,), jnp.float32),
            jax.ShapeDtypeStruct((bsz * n_per_batch,), jnp.int32),
        ),
        scratch_types=[
            pltpu.VMEM((npt,), jnp.float32),
            pltpu.VMEM((bpc * npt,), jnp.int32),
            pltpu.VMEM((ns * 256,), jnp.int32),
            pltpu.VMEM((hb,), jnp.int32),
            pltpu.VMEM_SHARED((ns * hb,), jnp.int32),
            pltpu.VMEM((ns * hb,), jnp.int32),
            pltpu.VMEM((npt,), jnp.float32),
            pltpu.VMEM((npt,), jnp.int32),
        ],
    )
    def sel(smap_hbm, scale_hbm, drop_hbm, x_v, u_v, hist, hsum, shared, gall,
            outs_v, outd_v):
        cid = lax.axis_index("c")
        sid = lax.axis_index("s")
        lane_iota = lax.iota(jnp.int32, 16)
        ones16 = jnp.ones((16,), jnp.int32)
        zeros16 = jnp.zeros((16,), jnp.int32)

        for i in range(bpc):
            off = (cid * bpc + i) * n_per_batch + sid * npt
            pltpu.sync_copy(smap_hbm.at[pl.ds(off, npt)], x_v)

            def stage(k, _):
                x = x_v[pl.ds(k * 16, 16)]
                u_v[pl.ds(i * npt + k * 16, 16)] = _u_of_bits(
                    lax.bitcast_convert_type(x, jnp.int32)
                )
                return 0

            lax.fori_loop(0, nv, stage, 0)

        prefix = [jnp.int32(0) for _ in range(bpc)]
        rank = [jnp.int32(num_samples) for _ in range(bpc)]

        for p in (3, 2, 1, 0):
            shift = 8 * p
            himask = jnp.int32(_to_i32(0xFFFFFFFF << (8 * (p + 1))))

            for i in range(bpc):
                def zero(j, _):
                    hist[pl.ds(j * 16, 16)] = zeros16
                    return 0

                lax.fori_loop(0, ns * 16, zero, 0)

                def scan(k, _):
                    u = u_v[pl.ds(i * npt + k * 16, 16)]
                    digit = lax.shift_right_logical(u, shift) & 255
                    grp = (u & himask) == prefix[i]
                    plsc.addupdate_scatter(
                        hist, [lane_iota * 256 + digit], ones16, mask=grp
                    )
                    return 0

                lax.fori_loop(0, nv, scan, 0)

                def lred_c(c, _):
                    def lred(l, acc):
                        return acc + hist[pl.ds(l * 256 + c * 16, 16)]

                    hsum[pl.ds(i * 256 + c * 16, 16)] = lax.fori_loop(
                        1, ns, lred, hist[pl.ds(c * 16, 16)]
                    )
                    return 0

                lax.fori_loop(0, 16, lred_c, 0)

            pltpu.sync_copy(hsum, shared.at[pl.ds(sid * hb, hb)])
            plsc.subcore_barrier()
            pltpu.sync_copy(shared, gall)
            plsc.subcore_barrier()

            for i in range(bpc):
                def tred_c(c, _):
                    def tred(t, acc):
                        return acc + gall[pl.ds(t * hb + i * 256 + c * 16, 16)]

                    hsum[pl.ds(i * 256 + c * 16, 16)] = lax.fori_loop(
                        1, ns, tred, gall[pl.ds(i * 256 + c * 16, 16)]
                    )
                    return 0

                lax.fori_loop(0, 16, tred_c, 0)

                def count_c(c, carry):
                    total_gt, running = carry
                    acc = hsum[pl.ds(i * 256 + c * 16, 16)]
                    cum = plsc.cumsum(acc) + running
                    gt = cum > rank[i]
                    return (
                        total_gt + jnp.sum(gt.astype(jnp.int32)),
                        jnp.max(cum),
                    )

                total_gt, _ = lax.fori_loop(
                    0, 16, count_c, (jnp.int32(0), jnp.int32(0))
                )
                digit = jnp.int32(256) - total_gt

                def below_c(c, below):
                    idx = lane_iota + c * 16
                    return below + jnp.sum(
                        jnp.where(
                            idx < digit, hsum[pl.ds(i * 256 + c * 16, 16)], 0
                        )
                    )

                below = lax.fori_loop(0, 16, below_c, jnp.int32(0))
                rank[i] = rank[i] - below
                prefix[i] = prefix[i] | lax.shift_left(digit, shift)

        for i in range(bpc):
            mthr = prefix[i] ^ jnp.int32(_MIN32)

            def emit(k, _):
                u = u_v[pl.ds(i * npt + k * 16, 16)]
                m = u ^ jnp.int32(_MIN32)
                keep = m > mthr
                outs_v[pl.ds(k * 16, 16)] = jnp.where(
                    keep, jnp.float32(_SCALE), jnp.float32(0.0)
                )
                outd_v[pl.ds(k * 16, 16)] = keep.astype(jnp.int32)
                return 0

            lax.fori_loop(0, nv, emit, 0)
            off = (cid * bpc + i) * n_per_batch + sid * npt
            pltpu.sync_copy(outs_v, scale_hbm.at[pl.ds(off, npt)])
            pltpu.sync_copy(outd_v, drop_hbm.at[pl.ds(off, npt)])

    return sel(smap_flat)


def _mask_body(img_ref, scale_ref, out_ref):
    out_ref[...] = img_ref[...] * scale_ref[...][:, None]


def kernel(image, saliency_map):
    bsz, chan, height, width = image.shape
    n = height * width
    num_samples = int(_DROP_PERCENT * height * width)

    scale_flat, drop_flat = _sc_select(
        num_samples, n, bsz, saliency_map.reshape(bsz * n)
    )
    scale_mask = scale_flat.reshape(bsz, height, width)
    drop = drop_flat.reshape(bsz, height, width).astype(bool)

    cb = 48
    masked = pl.pallas_call(
        _mask_body,
        grid=(bsz, chan // cb),
        in_specs=[
            pl.BlockSpec((1, cb, height, width), lambda b, c: (b, c, 0, 0)),
            pl.BlockSpec((1, height, width), lambda b, c: (b, 0, 0)),
        ],
        out_specs=pl.BlockSpec((1, cb, height, width), lambda b, c: (b, c, 0, 0)),
        out_shape=jax.ShapeDtypeStruct((bsz, chan, height, width), jnp.float32),
    )(image, scale_mask)

    return masked, drop


# speedup vs baseline: 1.0543x; 1.0543x over previous
import functools

import jax
import jax.numpy as jnp
from jax import lax
from jax.experimental import pallas as pl
from jax.experimental.pallas import tpu as pltpu
from jax.experimental.pallas import tpu_sc as plsc

_KEEP_PERCENT = 0.1
_SCALE = 1.0 / _KEEP_PERCENT
_DROP_PERCENT = 1.0 - _KEEP_PERCENT
_MIN32 = -(2 ** 31)
_LOW31 = 0x7FFFFFFF


def _to_i32(v):
    v &= 0xFFFFFFFF
    return v - 2 ** 32 if v >= 2 ** 31 else v


def _u_of_bits(b):
    m = b ^ (lax.shift_right_arithmetic(b, 31) & _LOW31)
    return m ^ jnp.int32(_MIN32)


def _sc_select(num_samples, n_per_batch, bsz, smap_flat):
    info = plsc.get_sparse_core_info()
    nc, ns = info.num_cores, info.num_subcores
    bpc = bsz // nc
    npt = n_per_batch // ns
    nv = npt // 16
    hw = ns * 256
    hb = bpc * 256
    mesh = plsc.VectorSubcoreMesh(core_axis_name="c", subcore_axis_name="s")

    @functools.partial(
        pl.kernel,
        mesh=mesh,
        compiler_params=pltpu.CompilerParams(needs_layout_passes=False),
        out_type=(
            jax.ShapeDtypeStruct((bsz * n_per_batch,), jnp.float32),
            jax.ShapeDtypeStruct((bsz * n_per_batch,), jnp.int32),
        ),
        scratch_types=[
            pltpu.VMEM((npt,), jnp.float32),
            pltpu.VMEM((bpc * npt,), jnp.int32),
            pltpu.VMEM((hw,), jnp.int32),
            pltpu.VMEM((hb,), jnp.int32),
            pltpu.VMEM_SHARED((ns * hb,), jnp.int32),
            pltpu.VMEM_SHARED((hw,), jnp.int32),
            pltpu.VMEM((ns * hb,), jnp.int32),
            pltpu.VMEM((npt,), jnp.float32),
            pltpu.VMEM((npt,), jnp.int32),
        ],
    )
    def sel(smap_hbm, scale_hbm, drop_hbm, x_v, u_v, hist, hsum, shared, zsh,
            gall, outs_v, outd_v):
        cid = lax.axis_index("c")
        sid = lax.axis_index("s")
        lane_iota = lax.iota(jnp.int32, 16)
        ones16 = jnp.ones((16,), jnp.int32)
        zeros16 = jnp.zeros((16,), jnp.int32)

        for i in range(bpc):
            off = (cid * bpc + i) * n_per_batch + sid * npt
            pltpu.sync_copy(smap_hbm.at[pl.ds(off, npt)], x_v)

            def stage(k, _):
                for q in range(4):
                    s = pl.ds(k * 64 + q * 16, 16)
                    u_v[pl.ds(i * npt + k * 64 + q * 16, 16)] = _u_of_bits(
                        lax.bitcast_convert_type(x_v[s], jnp.int32)
                    )
                return 0

            lax.fori_loop(0, nv // 4, stage, 0)

        def zero(j, _):
            for q in range(4):
                hist[pl.ds(j * 64 + q * 16, 16)] = zeros16
            return 0

        lax.fori_loop(0, hw // 64, zero, 0)

        @pl.when(sid == 0)
        def _():
            pltpu.sync_copy(hist, zsh)

        plsc.subcore_barrier()

        prefix = [jnp.int32(0) for _ in range(bpc)]
        rank = [jnp.int32(num_samples) for _ in range(bpc)]

        for p in (3, 2, 1, 0):
            shift = 8 * p
            himask = jnp.int32(_to_i32(0xFFFFFFFF << (8 * (p + 1))))

            for i in range(bpc):
                if p != 3:
                    pltpu.sync_copy(zsh, hist)

                def scan(k, _):
                    for q in range(4):
                        u = u_v[pl.ds(i * npt + k * 64 + q * 16, 16)]
                        digit = lax.shift_right_logical(u, shift) & 255
                        grp = (u & himask) == prefix[i]
                        plsc.addupdate_scatter(
                            hist, [lane_iota * 256 + digit], ones16, mask=grp
                        )
                    return 0

                lax.fori_loop(0, nv // 4, scan, 0)

                def lred_c(c, _):
                    acc = hist[pl.ds(c * 16, 16)]
                    for l in range(1, ns):
                        acc = acc + hist[pl.ds(l * 256 + c * 16, 16)]
                    hsum[pl.ds(i * 256 + c * 16, 16)] = acc
                    return 0

                lax.fori_loop(0, 16, lred_c, 0)

            pltpu.sync_copy(hsum, shared.at[pl.ds(sid * hb, hb)])
            plsc.subcore_barrier()
            pltpu.sync_copy(shared, gall)
            plsc.subcore_barrier()

            for i in range(bpc):
                def tred_c(c, _):
                    acc = gall[pl.ds(i * 256 + c * 16, 16)]
                    for t in range(1, ns):
                        acc = acc + gall[pl.ds(t * hb + i * 256 + c * 16, 16)]
                    hsum[pl.ds(i * 256 + c * 16, 16)] = acc
                    return 0

                lax.fori_loop(0, 16, tred_c, 0)

                def count_c(c, carry):
                    total_gt, running = carry
                    acc = hsum[pl.ds(i * 256 + c * 16, 16)]
                    cum = plsc.cumsum(acc) + running
                    gt = cum > rank[i]
                    return (
                        total_gt + jnp.sum(gt.astype(jnp.int32)),
                        jnp.max(cum),
                    )

                total_gt, _ = lax.fori_loop(
                    0, 16, count_c, (jnp.int32(0), jnp.int32(0))
                )
                digit = jnp.int32(256) - total_gt

                def below_c(c, below):
                    idx = lane_iota + c * 16
                    return below + jnp.sum(
                        jnp.where(
                            idx < digit, hsum[pl.ds(i * 256 + c * 16, 16)], 0
                        )
                    )

                below = lax.fori_loop(0, 16, below_c, jnp.int32(0))
                rank[i] = rank[i] - below
                prefix[i] = prefix[i] | lax.shift_left(digit, shift)

        for i in range(bpc):
            mthr = prefix[i] ^ jnp.int32(_MIN32)

            def emit(k, _):
                for q in range(4):
                    s = pl.ds(k * 64 + q * 16, 16)
                    u = u_v[pl.ds(i * npt + k * 64 + q * 16, 16)]
                    keep = (u ^ jnp.int32(_MIN32)) > mthr
                    outs_v[s] = jnp.where(
                        keep, jnp.float32(_SCALE), jnp.float32(0.0)
                    )
                    outd_v[s] = keep.astype(jnp.int32)
                return 0

            lax.fori_loop(0, nv // 4, emit, 0)
            off = (cid * bpc + i) * n_per_batch + sid * npt
            pltpu.sync_copy(outs_v, scale_hbm.at[pl.ds(off, npt)])
            pltpu.sync_copy(outd_v, drop_hbm.at[pl.ds(off, npt)])

    return sel(smap_flat)


def _mask_body(img_ref, scale_ref, out_ref):
    out_ref[...] = img_ref[...] * scale_ref[...][:, None]


def kernel(image, saliency_map):
    bsz, chan, height, width = image.shape
    n = height * width
    num_samples = int(_DROP_PERCENT * height * width)

    scale_flat, drop_flat = _sc_select(
        num_samples, n, bsz, saliency_map.reshape(bsz * n)
    )
    scale_mask = scale_flat.reshape(bsz, height, width)
    drop = drop_flat.reshape(bsz, height, width).astype(bool)

    cb = 48
    masked = pl.pallas_call(
        _mask_body,
        grid=(bsz, chan // cb),
        in_specs=[
            pl.BlockSpec((1, cb, height, width), lambda b, c: (b, c, 0, 0)),
            pl.BlockSpec((1, height, width), lambda b, c: (b, 0, 0)),
        ],
        out_specs=pl.BlockSpec((1, cb, height, width), lambda b, c: (b, c, 0, 0)),
        out_shape=jax.ShapeDtypeStruct((bsz, chan, height, width), jnp.float32),
    )(image, scale_mask)

    return masked, drop


# speedup vs baseline: 1.0618x; 1.0071x over previous
import functools

import jax
import jax.numpy as jnp
from jax import lax
from jax.experimental import pallas as pl
from jax.experimental.pallas import tpu as pltpu
from jax.experimental.pallas import tpu_sc as plsc

_KEEP_PERCENT = 0.1
_SCALE = 1.0 / _KEEP_PERCENT
_DROP_PERCENT = 1.0 - _KEEP_PERCENT
_MIN32 = -(2 ** 31)
_LOW31 = 0x7FFFFFFF


def _to_i32(v):
    v &= 0xFFFFFFFF
    return v - 2 ** 32 if v >= 2 ** 31 else v


def _u_of_bits(b):
    m = b ^ (lax.shift_right_arithmetic(b, 31) & _LOW31)
    return m ^ jnp.int32(_MIN32)


def _sc_select(num_samples, n_per_batch, bsz, smap_flat):
    info = plsc.get_sparse_core_info()
    nc, ns = info.num_cores, info.num_subcores
    bpc = bsz // nc
    npt = n_per_batch // ns
    nv = npt // 16
    hw = ns * 256
    hb = bpc * 256
    mesh = plsc.VectorSubcoreMesh(core_axis_name="c", subcore_axis_name="s")

    @functools.partial(
        pl.kernel,
        mesh=mesh,
        compiler_params=pltpu.CompilerParams(needs_layout_passes=False),
        out_type=(
            jax.ShapeDtypeStruct((bsz * n_per_batch,), jnp.float32),
            jax.ShapeDtypeStruct((bsz * n_per_batch,), jnp.int32),
        ),
        scratch_types=[
            pltpu.VMEM((npt,), jnp.float32),
            pltpu.VMEM((bpc * npt,), jnp.int32),
            pltpu.VMEM((hw,), jnp.int32),
            pltpu.VMEM((hb,), jnp.int32),
            pltpu.VMEM_SHARED((ns * hb,), jnp.int32),
            pltpu.VMEM_SHARED((hw,), jnp.int32),
            pltpu.VMEM((ns * hb,), jnp.int32),
            pltpu.VMEM((npt,), jnp.float32),
            pltpu.VMEM((npt,), jnp.int32),
        ],
    )
    def sel(smap_hbm, scale_hbm, drop_hbm, x_v, u_v, hist, hsum, shared, zsh,
            gall, outs_v, outd_v):
        cid = lax.axis_index("c")
        sid = lax.axis_index("s")
        lane_iota = lax.iota(jnp.int32, 16)
        ones16 = jnp.ones((16,), jnp.int32)
        zeros16 = jnp.zeros((16,), jnp.int32)

        for i in range(bpc):
            off = (cid * bpc + i) * n_per_batch + sid * npt
            pltpu.sync_copy(smap_hbm.at[pl.ds(off, npt)], x_v)

            def stage(k, _):
                for q in range(4):
                    s = pl.ds(k * 64 + q * 16, 16)
                    u_v[pl.ds(i * npt + k * 64 + q * 16, 16)] = _u_of_bits(
                        lax.bitcast_convert_type(x_v[s], jnp.int32)
                    )
                return 0

            lax.fori_loop(0, nv // 4, stage, 0)

        def zero(j, _):
            for q in range(4):
                hist[pl.ds(j * 64 + q * 16, 16)] = zeros16
            return 0

        lax.fori_loop(0, hw // 64, zero, 0)

        @pl.when(sid == 0)
        def _():
            pltpu.sync_copy(hist, zsh)

        plsc.subcore_barrier()

        prefix = [jnp.int32(0) for _ in range(bpc)]
        rank = [jnp.int32(num_samples) for _ in range(bpc)]

        for p in (3, 2, 1, 0):
            shift = 8 * p
            himask = jnp.int32(_to_i32(0xFFFFFFFF << (8 * (p + 1))))

            for i in range(bpc):
                if not (p == 3 and i == 0):
                    pltpu.sync_copy(zsh, hist)

                def scan(k, _):
                    for q in range(4):
                        u = u_v[pl.ds(i * npt + k * 64 + q * 16, 16)]
                        digit = lax.shift_right_logical(u, shift) & 255
                        grp = (u & himask) == prefix[i]
                        plsc.addupdate_scatter(
                            hist, [lane_iota * 256 + digit], ones16, mask=grp
                        )
                    return 0

                lax.fori_loop(0, nv // 4, scan, 0)

                def lred_c(c, _):
                    acc = hist[pl.ds(c * 16, 16)]
                    for l in range(1, ns):
                        acc = acc + hist[pl.ds(l * 256 + c * 16, 16)]
                    hsum[pl.ds(i * 256 + c * 16, 16)] = acc
                    return 0

                lax.fori_loop(0, 16, lred_c, 0)

            pltpu.sync_copy(hsum, shared.at[pl.ds(sid * hb, hb)])
            plsc.subcore_barrier()
            pltpu.sync_copy(shared, gall)
            plsc.subcore_barrier()

            for i in range(bpc):
                def count_c(c, carry):
                    total_gt, running = carry
                    acc = gall[pl.ds(i * 256 + c * 16, 16)]
                    for t in range(1, ns):
                        acc = acc + gall[pl.ds(t * hb + i * 256 + c * 16, 16)]
                    hsum[pl.ds(i * 256 + c * 16, 16)] = acc
                    cum = plsc.cumsum(acc) + running
                    gt = cum > rank[i]
                    return (
                        total_gt + jnp.sum(gt.astype(jnp.int32)),
                        jnp.max(cum),
                    )

                total_gt, _ = lax.fori_loop(
                    0, 16, count_c, (jnp.int32(0), jnp.int32(0))
                )
                digit = jnp.int32(256) - total_gt

                def below_c(c, below):
                    idx = lane_iota + c * 16
                    return below + jnp.sum(
                        jnp.where(
                            idx < digit, hsum[pl.ds(i * 256 + c * 16, 16)], 0
                        )
                    )

                below = lax.fori_loop(0, 16, below_c, jnp.int32(0))
                rank[i] = rank[i] - below
                prefix[i] = prefix[i] | lax.shift_left(digit, shift)

        for i in range(bpc):
            mthr = prefix[i] ^ jnp.int32(_MIN32)

            def emit(k, _):
                for q in range(4):
                    s = pl.ds(k * 64 + q * 16, 16)
                    u = u_v[pl.ds(i * npt + k * 64 + q * 16, 16)]
                    keep = (u ^ jnp.int32(_MIN32)) > mthr
                    outs_v[s] = jnp.where(
                        keep, jnp.float32(_SCALE), jnp.float32(0.0)
                    )
                    outd_v[s] = keep.astype(jnp.int32)
                return 0

            lax.fori_loop(0, nv // 4, emit, 0)
            off = (cid * bpc + i) * n_per_batch + sid * npt
            pltpu.sync_copy(outs_v, scale_hbm.at[pl.ds(off, npt)])
            pltpu.sync_copy(outd_v, drop_hbm.at[pl.ds(off, npt)])

    return sel(smap_flat)


def _mask_body(img_ref, scale_ref, out_ref):
    out_ref[...] = img_ref[...] * scale_ref[...][:, None]


def kernel(image, saliency_map):
    bsz, chan, height, width = image.shape
    n = height * width
    num_samples = int(_DROP_PERCENT * height * width)

    scale_flat, drop_flat = _sc_select(
        num_samples, n, bsz, saliency_map.reshape(bsz * n)
    )
    scale_mask = scale_flat.reshape(bsz, height, width)
    drop = drop_flat.reshape(bsz, height, width).astype(bool)

    cb = 48
    masked = pl.pallas_call(
        _mask_body,
        grid=(bsz, chan // cb),
        in_specs=[
            pl.BlockSpec((1, cb, height, width), lambda b, c: (b, c, 0, 0)),
            pl.BlockSpec((1, height, width), lambda b, c: (b, 0, 0)),
        ],
        out_specs=pl.BlockSpec((1, cb, height, width), lambda b, c: (b, c, 0, 0)),
        out_shape=jax.ShapeDtypeStruct((bsz, chan, height, width), jnp.float32),
    )(image, scale_mask)

    return masked, drop


# speedup vs baseline: 1.0664x; 1.0043x over previous
import functools

import jax
import jax.numpy as jnp
from jax import lax
from jax.experimental import pallas as pl
from jax.experimental.pallas import tpu as pltpu
from jax.experimental.pallas import tpu_sc as plsc

_KEEP_PERCENT = 0.1
_SCALE = 1.0 / _KEEP_PERCENT
_DROP_PERCENT = 1.0 - _KEEP_PERCENT
_MIN32 = -(2 ** 31)
_LOW31 = 0x7FFFFFFF


def _to_i32(v):
    v &= 0xFFFFFFFF
    return v - 2 ** 32 if v >= 2 ** 31 else v


def _u_of_bits(b):
    m = b ^ (lax.shift_right_arithmetic(b, 31) & _LOW31)
    return m ^ jnp.int32(_MIN32)


def _sc_select(num_samples, n_per_batch, bsz, smap_flat):
    info = plsc.get_sparse_core_info()
    nc, ns = info.num_cores, info.num_subcores
    bpc = bsz // nc
    npt = n_per_batch // ns
    nv = npt // 16
    hstr = 257
    hw = ((ns * hstr + 63) // 64) * 64
    hb = bpc * 256
    mesh = plsc.VectorSubcoreMesh(core_axis_name="c", subcore_axis_name="s")

    @functools.partial(
        pl.kernel,
        mesh=mesh,
        compiler_params=pltpu.CompilerParams(needs_layout_passes=False),
        out_type=(
            jax.ShapeDtypeStruct((bsz * n_per_batch,), jnp.float32),
            jax.ShapeDtypeStruct((bsz * n_per_batch,), jnp.int32),
        ),
        scratch_types=[
            pltpu.VMEM((npt,), jnp.float32),
            pltpu.VMEM((bpc * npt,), jnp.int32),
            pltpu.VMEM((hw,), jnp.int32),
            pltpu.VMEM((hb,), jnp.int32),
            pltpu.VMEM_SHARED((ns * hb,), jnp.int32),
            pltpu.VMEM_SHARED((hw,), jnp.int32),
            pltpu.VMEM_SHARED((16,), jnp.int32),
            pltpu.VMEM((ns * hb,), jnp.int32),
            pltpu.VMEM((16,), jnp.int32),
            pltpu.VMEM((npt,), jnp.float32),
            pltpu.VMEM((npt,), jnp.int32),
        ],
    )
    def sel(smap_hbm, scale_hbm, drop_hbm, x_v, u_v, hist, hsum, shared, zsh,
            pub_sp, gall, pubv, outs_v, outd_v):
        cid = lax.axis_index("c")
        sid = lax.axis_index("s")
        lane_iota = lax.iota(jnp.int32, 16)
        ones16 = jnp.ones((16,), jnp.int32)
        zeros16 = jnp.zeros((16,), jnp.int32)

        for i in range(bpc):
            off = (cid * bpc + i) * n_per_batch + sid * npt
            pltpu.sync_copy(smap_hbm.at[pl.ds(off, npt)], x_v)

            def stage(k, _):
                for q in range(4):
                    s = pl.ds(k * 64 + q * 16, 16)
                    u_v[pl.ds(i * npt + k * 64 + q * 16, 16)] = _u_of_bits(
                        lax.bitcast_convert_type(x_v[s], jnp.int32)
                    )
                return 0

            lax.fori_loop(0, nv // 4, stage, 0)

        def zero(j, _):
            for q in range(4):
                hist[pl.ds(j * 64 + q * 16, 16)] = zeros16
            return 0

        lax.fori_loop(0, hw // 64, zero, 0)

        @pl.when(sid == 0)
        def _():
            pltpu.sync_copy(hist, zsh)

        plsc.subcore_barrier()

        prefix = [jnp.int32(0) for _ in range(bpc)]
        rank = [jnp.int32(num_samples) for _ in range(bpc)]

        for p in (3, 2, 1, 0):
            shift = 8 * p
            himask = jnp.int32(_to_i32(0xFFFFFFFF << (8 * (p + 1))))

            for i in range(bpc):
                if not (p == 3 and i == 0):
                    pltpu.sync_copy(zsh, hist)

                def scan(k, _):
                    for q in range(4):
                        u = u_v[pl.ds(i * npt + k * 64 + q * 16, 16)]
                        digit = lax.shift_right_logical(u, shift) & 255
                        grp = (u & himask) == prefix[i]
                        plsc.addupdate_scatter(
                            hist, [lane_iota * hstr + digit], ones16, mask=grp
                        )
                    return 0

                lax.fori_loop(0, nv // 4, scan, 0)

                def lred_c(c, _):
                    acc = hist[pl.ds(c * 16, 16)]
                    for l in range(1, ns):
                        acc = acc + hist[pl.ds(l * hstr + c * 16, 16)]
                    hsum[pl.ds(i * 256 + c * 16, 16)] = acc
                    return 0

                lax.fori_loop(0, 16, lred_c, 0)

            pltpu.sync_copy(hsum, shared.at[pl.ds(sid * hb, hb)])
            plsc.subcore_barrier()

            @pl.when(sid == 0)
            def _():
                pltpu.sync_copy(shared, gall)
                pub = jnp.zeros((16,), jnp.int32)
                for i in range(bpc):
                    def count_c(c, carry):
                        total_gt, running = carry
                        acc = gall[pl.ds(i * 256 + c * 16, 16)]
                        for t in range(1, ns):
                            acc = acc + gall[
                                pl.ds(t * hb + i * 256 + c * 16, 16)
                            ]
                        hsum[pl.ds(i * 256 + c * 16, 16)] = acc
                        cum = plsc.cumsum(acc) + running
                        gt = cum > rank[i]
                        return (
                            total_gt + jnp.sum(gt.astype(jnp.int32)),
                            jnp.max(cum),
                        )

                    total_gt, _ = lax.fori_loop(
                        0, 16, count_c, (jnp.int32(0), jnp.int32(0))
                    )
                    digit = jnp.int32(256) - total_gt

                    def below_c(c, below):
                        idx = lane_iota + c * 16
                        return below + jnp.sum(
                            jnp.where(
                                idx < digit,
                                hsum[pl.ds(i * 256 + c * 16, 16)],
                                0,
                            )
                        )

                    below = lax.fori_loop(0, 16, below_c, jnp.int32(0))
                    pub = jnp.where(lane_iota == i, digit, pub)
                    pub = jnp.where(lane_iota == 8 + i, below, pub)
                pubv[...] = pub
                pltpu.sync_copy(pubv, pub_sp)

            plsc.subcore_barrier()
            pltpu.sync_copy(pub_sp, pubv)
            pv = pubv[...]
            for i in range(bpc):
                digit = jnp.sum(jnp.where(lane_iota == i, pv, 0))
                below = jnp.sum(jnp.where(lane_iota == 8 + i, pv, 0))
                rank[i] = rank[i] - below
                prefix[i] = prefix[i] | lax.shift_left(digit, shift)

        for i in range(bpc):
            mthr = prefix[i] ^ jnp.int32(_MIN32)

            def emit(k, _):
                for q in range(4):
                    s = pl.ds(k * 64 + q * 16, 16)
                    u = u_v[pl.ds(i * npt + k * 64 + q * 16, 16)]
                    keep = (u ^ jnp.int32(_MIN32)) > mthr
                    outs_v[s] = jnp.where(
                        keep, jnp.float32(_SCALE), jnp.float32(0.0)
                    )
                    outd_v[s] = keep.astype(jnp.int32)
                return 0

            lax.fori_loop(0, nv // 4, emit, 0)
            off = (cid * bpc + i) * n_per_batch + sid * npt
            pltpu.sync_copy(outs_v, scale_hbm.at[pl.ds(off, npt)])
            pltpu.sync_copy(outd_v, drop_hbm.at[pl.ds(off, npt)])

    return sel(smap_flat)


def _mask_body(img_ref, scale_ref, out_ref):
    out_ref[...] = img_ref[...] * scale_ref[...][:, None]


def kernel(image, saliency_map):
    bsz, chan, height, width = image.shape
    n = height * width
    num_samples = int(_DROP_PERCENT * height * width)

    scale_flat, drop_flat = _sc_select(
        num_samples, n, bsz, saliency_map.reshape(bsz * n)
    )
    scale_mask = scale_flat.reshape(bsz, height, width)
    drop = drop_flat.reshape(bsz, height, width).astype(bool)

    cb = 48
    masked = pl.pallas_call(
        _mask_body,
        grid=(bsz, chan // cb),
        in_specs=[
            pl.BlockSpec((1, cb, height, width), lambda b, c: (b, c, 0, 0)),
            pl.BlockSpec((1, height, width), lambda b, c: (b, 0, 0)),
        ],
        out_specs=pl.BlockSpec((1, cb, height, width), lambda b, c: (b, c, 0, 0)),
        out_shape=jax.ShapeDtypeStruct((bsz, chan, height, width), jnp.float32),
    )(image, scale_mask)

    return masked, drop


# speedup vs baseline: 1.0937x; 1.0256x over previous
import functools

import jax
import jax.numpy as jnp
from jax import lax
from jax.experimental import pallas as pl
from jax.experimental.pallas import tpu as pltpu
from jax.experimental.pallas import tpu_sc as plsc

_KEEP_PERCENT = 0.1
_SCALE = 1.0 / _KEEP_PERCENT
_DROP_PERCENT = 1.0 - _KEEP_PERCENT
_MIN32 = -(2 ** 31)
_LOW31 = 0x7FFFFFFF


def _to_i32(v):
    v &= 0xFFFFFFFF
    return v - 2 ** 32 if v >= 2 ** 31 else v


def _u_of_bits(b):
    m = b ^ (lax.shift_right_arithmetic(b, 31) & _LOW31)
    return m ^ jnp.int32(_MIN32)


def _sc_select(num_samples, n_per_batch, bsz, smap_flat):
    info = plsc.get_sparse_core_info()
    nc, ns = info.num_cores, info.num_subcores
    bpc = bsz // nc
    npt = n_per_batch // ns
    nv = npt // 16
    hstr = 257
    hw = ((ns * hstr + 63) // 64) * 64
    hb = bpc * 256
    mesh = plsc.VectorSubcoreMesh(core_axis_name="c", subcore_axis_name="s")

    @functools.partial(
        pl.kernel,
        mesh=mesh,
        compiler_params=pltpu.CompilerParams(needs_layout_passes=False),
        out_type=jax.ShapeDtypeStruct((nc * 16,), jnp.float32),
        scratch_types=[
            pltpu.VMEM((npt,), jnp.float32),
            pltpu.VMEM((bpc * npt,), jnp.int32),
            pltpu.VMEM((hw,), jnp.int32),
            pltpu.VMEM((hb,), jnp.int32),
            pltpu.VMEM_SHARED((ns * hb,), jnp.int32),
            pltpu.VMEM_SHARED((hw,), jnp.int32),
            pltpu.VMEM_SHARED((16,), jnp.int32),
            pltpu.VMEM((ns * hb,), jnp.int32),
            pltpu.VMEM((16,), jnp.int32),
            pltpu.VMEM((16,), jnp.float32),
        ],
    )
    def sel(smap_hbm, thr_hbm, x_v, u_v, hist, hsum, shared, zsh, pub_sp,
            gall, pubv, thrv):
        cid = lax.axis_index("c")
        sid = lax.axis_index("s")
        lane_iota = lax.iota(jnp.int32, 16)
        ones16 = jnp.ones((16,), jnp.int32)
        zeros16 = jnp.zeros((16,), jnp.int32)

        for i in range(bpc):
            off = (cid * bpc + i) * n_per_batch + sid * npt
            pltpu.sync_copy(smap_hbm.at[pl.ds(off, npt)], x_v)

            def stage(k, _):
                for q in range(4):
                    s = pl.ds(k * 64 + q * 16, 16)
                    u_v[pl.ds(i * npt + k * 64 + q * 16, 16)] = _u_of_bits(
                        lax.bitcast_convert_type(x_v[s], jnp.int32)
                    )
                return 0

            lax.fori_loop(0, nv // 4, stage, 0)

        def zero(j, _):
            for q in range(4):
                hist[pl.ds(j * 64 + q * 16, 16)] = zeros16
            return 0

        lax.fori_loop(0, hw // 64, zero, 0)

        @pl.when(sid == 0)
        def _():
            pltpu.sync_copy(hist, zsh)

        plsc.subcore_barrier()

        prefix = [jnp.int32(0) for _ in range(bpc)]
        rank = [jnp.int32(num_samples) for _ in range(bpc)]

        for p in (3, 2, 1, 0):
            shift = 8 * p
            himask = jnp.int32(_to_i32(0xFFFFFFFF << (8 * (p + 1))))

            for i in range(bpc):
                if not (p == 3 and i == 0):
                    pltpu.sync_copy(zsh, hist)

                def scan(k, _):
                    for q in range(4):
                        u = u_v[pl.ds(i * npt + k * 64 + q * 16, 16)]
                        digit = lax.shift_right_logical(u, shift) & 255
                        grp = (u & himask) == prefix[i]
                        plsc.addupdate_scatter(
                            hist, [lane_iota * hstr + digit], ones16, mask=grp
                        )
                    return 0

                lax.fori_loop(0, nv // 4, scan, 0)

                def lred_c(c, _):
                    acc = hist[pl.ds(c * 16, 16)]
                    for l in range(1, ns):
                        acc = acc + hist[pl.ds(l * hstr + c * 16, 16)]
                    hsum[pl.ds(i * 256 + c * 16, 16)] = acc
                    return 0

                lax.fori_loop(0, 16, lred_c, 0)

            pltpu.sync_copy(hsum, shared.at[pl.ds(sid * hb, hb)])
            plsc.subcore_barrier()

            @pl.when(sid == 0)
            def _():
                pltpu.sync_copy(shared, gall)
                pub = jnp.zeros((16,), jnp.int32)
                for i in range(bpc):
                    def count_c(c, carry):
                        total_gt, running = carry
                        acc = gall[pl.ds(i * 256 + c * 16, 16)]
                        for t in range(1, ns):
                            acc = acc + gall[
                                pl.ds(t * hb + i * 256 + c * 16, 16)
                            ]
                        hsum[pl.ds(i * 256 + c * 16, 16)] = acc
                        cum = plsc.cumsum(acc) + running
                        gt = cum > rank[i]
                        return (
                            total_gt + jnp.sum(gt.astype(jnp.int32)),
                            jnp.max(cum),
                        )

                    total_gt, _ = lax.fori_loop(
                        0, 16, count_c, (jnp.int32(0), jnp.int32(0))
                    )
                    digit = jnp.int32(256) - total_gt

                    def below_c(c, below):
                        idx = lane_iota + c * 16
                        return below + jnp.sum(
                            jnp.where(
                                idx < digit,
                                hsum[pl.ds(i * 256 + c * 16, 16)],
                                0,
                            )
                        )

                    below = lax.fori_loop(0, 16, below_c, jnp.int32(0))
                    pub = jnp.where(lane_iota == i, digit, pub)
                    pub = jnp.where(lane_iota == 8 + i, below, pub)
                pubv[...] = pub
                pltpu.sync_copy(pubv, pub_sp)

            plsc.subcore_barrier()
            pltpu.sync_copy(pub_sp, pubv)
            pv = pubv[...]
            for i in range(bpc):
                digit = jnp.sum(jnp.where(lane_iota == i, pv, 0))
                below = jnp.sum(jnp.where(lane_iota == 8 + i, pv, 0))
                rank[i] = rank[i] - below
                prefix[i] = prefix[i] | lax.shift_left(digit, shift)

        @pl.when(sid == 0)
        def _():
            tf = jnp.zeros((16,), jnp.float32)
            for i in range(bpc):
                m = prefix[i] ^ jnp.int32(_MIN32)
                fb = m ^ (lax.shift_right_arithmetic(m, 31) & _LOW31)
                fv = lax.bitcast_convert_type(
                    jnp.full((16,), fb, jnp.int32), jnp.float32
                )
                tf = jnp.where(lane_iota == i, fv, tf)
            thrv[...] = tf
            pltpu.sync_copy(thrv, thr_hbm.at[pl.ds(cid * 16, 16)])

    return sel(smap_flat)


def _mask_body(img_ref, smap_ref, thr_ref, out_ref, drop_ref):
    b = pl.program_id(0)
    c = pl.program_id(1)
    t = thr_ref[16 * (b // 2) + (b % 2)]
    keep = smap_ref[...] > t
    scale = jnp.where(keep, jnp.float32(_SCALE), jnp.float32(0.0))
    out_ref[...] = img_ref[...] * scale[:, None]

    @pl.when(c == 0)
    def _():
        drop_ref[...] = keep.astype(jnp.int32)


def kernel(image, saliency_map):
    bsz, chan, height, width = image.shape
    n = height * width
    num_samples = int(_DROP_PERCENT * height * width)

    thr = _sc_select(num_samples, n, bsz, saliency_map.reshape(bsz * n))

    cb = 48
    masked, drop = pl.pallas_call(
        _mask_body,
        grid=(bsz, chan // cb),
        in_specs=[
            pl.BlockSpec((1, cb, height, width), lambda b, c: (b, c, 0, 0)),
            pl.BlockSpec((1, height, width), lambda b, c: (b, 0, 0)),
            pl.BlockSpec(memory_space=pltpu.SMEM),
        ],
        out_specs=[
            pl.BlockSpec((1, cb, height, width), lambda b, c: (b, c, 0, 0)),
            pl.BlockSpec((1, height, width), lambda b, c: (b, 0, 0)),
        ],
        out_shape=(
            jax.ShapeDtypeStruct((bsz, chan, height, width), jnp.float32),
            jax.ShapeDtypeStruct((bsz, height, width), jnp.int32),
        ),
    )(image, saliency_map, thr)

    return masked, drop.astype(bool)
